# Initial kernel scaffold; baseline (speedup 1.0000x reference)
import jax
import jax.numpy as jnp
from jax.experimental import pallas as pl


def kernel(tree_embedding, node_connection, node_mask, W_f, b_f, W_o, b_o, W_z, b_z, T_f, T_o, T_z, init_h, init_c):
    raise NotImplementedError("write your pallas kernel here")


# speedup vs baseline: 8.5346x; 8.5346x over previous
import functools

import jax
import jax.numpy as jnp
from jax.experimental import pallas as pl
from jax.experimental.pallas import tpu as pltpu

BATCH = 1024
NODE_NUM = 128
INPUT_SZ = 128
HIDDEN_SZ = 128

BB = 128


def _kernel_body(emb0_ref, connT_ref, Wcat_ref, bcat_ref, Tf_ref, To_ref,
                 Tz_ref, init_h_ref, init_c_ref, out_ref, c_ref, gh_ref,
                 gc_ref):
    H = HIDDEN_SZ
    prec = jax.lax.Precision.HIGHEST

    def tt(t_ref):
        t = t_ref[...]
        return jax.lax.dot_general(t, t, (((0,), (0,)), ((), ())),
                                   precision=prec,
                                   preferred_element_type=jnp.float32)

    vcat = jnp.concatenate([tt(Tf_ref), tt(To_ref), tt(Tz_ref)], axis=1)

    foz0 = jax.lax.dot_general(emb0_ref[...], Wcat_ref[...],
                               (((1,), (0,)), ((), ())), precision=prec,
                               preferred_element_type=jnp.float32)
    foz0 = foz0 + bcat_ref[...]
    f0 = foz0[:, 0:H]
    o0 = foz0[:, H:2 * H]
    z0 = foz0[:, 2 * H:3 * H]

    f = jax.nn.sigmoid(f0)
    o = jax.nn.sigmoid(o0)
    z = jnp.tanh(z0)
    c0 = z * (1.0 - f)
    h0 = o * jnp.tanh(c0)

    out_ref[...] = init_h_ref[...]
    c_ref[...] = init_c_ref[...]
    out_ref[:, 0, :] = h0
    c_ref[:, 0, :] = c0

    def step(i, _):
        def gather_row(b, _):
            p = connT_ref[i, b]
            gh_ref[pl.ds(b, 1), :] = out_ref[pl.ds(b, 1), pl.ds(p, 1),
                                             :].reshape(1, H)
            gc_ref[pl.ds(b, 1), :] = c_ref[pl.ds(b, 1), pl.ds(p, 1),
                                           :].reshape(1, H)
            return 0

        jax.lax.fori_loop(0, BB, gather_row, 0, unroll=8)

        gates = jax.lax.dot_general(gh_ref[...], vcat, (((1,), (0,)), ((), ())),
                                    precision=prec,
                                    preferred_element_type=jnp.float32)
        gates = gates + foz0
        f = jax.nn.sigmoid(gates[:, 0:H])
        o = jax.nn.sigmoid(gates[:, H:2 * H])
        z = jnp.tanh(gates[:, 2 * H:3 * H])
        c = gc_ref[...] * f + z * (1.0 - f)
        h = o * jnp.tanh(c)
        out_ref[:, pl.ds(i, 1), :] = h[:, None, :]
        c_ref[:, pl.ds(i, 1), :] = c[:, None, :]
        return 0

    jax.lax.fori_loop(1, NODE_NUM, step, 0)


def kernel(tree_embedding, node_connection, node_mask, W_f, b_f, W_o, b_o,
           W_z, b_z, T_f, T_o, T_z, init_h, init_c):
    del node_mask
    H = HIDDEN_SZ
    emb0 = tree_embedding[:, 0, :]
    connT = jnp.swapaxes(node_connection, 0, 1)
    Wcat = jnp.concatenate([W_f.T, W_o.T, W_z.T], axis=1)
    bcat = jnp.concatenate([b_f, b_o, b_z]).reshape(1, 3 * H)

    nb = BATCH // BB
    grid = (nb,)
    out = pl.pallas_call(
        _kernel_body,
        grid=grid,
        in_specs=[
            pl.BlockSpec((BB, INPUT_SZ), lambda j: (j, 0)),
            pl.BlockSpec((NODE_NUM, BB), lambda j: (0, j),
                         memory_space=pltpu.SMEM),
            pl.BlockSpec((INPUT_SZ, 3 * H), lambda j: (0, 0)),
            pl.BlockSpec((1, 3 * H), lambda j: (0, 0)),
            pl.BlockSpec((H, H), lambda j: (0, 0)),
            pl.BlockSpec((H, H), lambda j: (0, 0)),
            pl.BlockSpec((H, H), lambda j: (0, 0)),
            pl.BlockSpec((BB, NODE_NUM, H), lambda j: (j, 0, 0)),
            pl.BlockSpec((BB, NODE_NUM, H), lambda j: (j, 0, 0)),
        ],
        out_specs=pl.BlockSpec((BB, NODE_NUM, H), lambda j: (j, 0, 0)),
        out_shape=jax.ShapeDtypeStruct((BATCH, NODE_NUM, H), jnp.float32),
        scratch_shapes=[
            pltpu.VMEM((BB, NODE_NUM, H), jnp.float32),
            pltpu.VMEM((BB, H), jnp.float32),
            pltpu.VMEM((BB, H), jnp.float32),
        ],
    )(emb0, connT, Wcat, bcat, T_f, T_o, T_z, init_h, init_c)
    return out


# speedup vs baseline: 9.4514x; 1.1074x over previous
import jax
import jax.numpy as jnp
from jax.experimental import pallas as pl
from jax.experimental.pallas import tpu as pltpu

BATCH = 1024
NODE_NUM = 128
INPUT_SZ = 128
HIDDEN_SZ = 128

BB = 128


def _kernel_body(emb0_ref, connT_ref, Wcat_ref, bcat_ref, Tf_ref, To_ref,
                 Tz_ref, init_ref, out_ref, state_ref, gbuf_ref, sem):
    H = HIDDEN_SZ
    prec = jax.lax.Precision.HIGHEST
    j = pl.program_id(0)

    cp = pltpu.make_async_copy(init_ref.at[:, pl.ds(j * BB, BB), :],
                               state_ref, sem)
    cp.start()

    def tt(t_ref):
        t = t_ref[...]
        return jax.lax.dot_general(t, t, (((0,), (0,)), ((), ())),
                                   precision=prec,
                                   preferred_element_type=jnp.float32)

    vcat = jnp.concatenate([tt(Tf_ref), tt(To_ref), tt(Tz_ref)], axis=1)

    foz0 = jax.lax.dot_general(emb0_ref[...], Wcat_ref[...],
                               (((1,), (0,)), ((), ())), precision=prec,
                               preferred_element_type=jnp.float32)
    foz0 = foz0 + bcat_ref[...]
    f0 = foz0[:, 0:H]
    o0 = foz0[:, H:2 * H]
    z0 = foz0[:, 2 * H:3 * H]

    f = jax.nn.sigmoid(f0)
    o = jax.nn.sigmoid(o0)
    z = jnp.tanh(z0)
    c0 = z * (1.0 - f)
    h0 = o * jnp.tanh(c0)

    cp.wait()
    state_ref[pl.ds(0, 1), :, 0:H] = h0[None]
    state_ref[pl.ds(0, 1), :, H:2 * H] = c0[None]
    out_ref[pl.ds(0, 1), :, :] = h0[None]

    def step(i, _):
        for b in range(BB):
            p = connT_ref[i, b]
            gbuf_ref[:, pl.ds(b, 1), :] = state_ref[pl.ds(p, 1),
                                                    pl.ds(b, 1), :]
        g = gbuf_ref[0]
        gh = g[:, 0:H]
        gc = g[:, H:2 * H]
        gates = jax.lax.dot_general(gh, vcat, (((1,), (0,)), ((), ())),
                                    precision=prec,
                                    preferred_element_type=jnp.float32)
        gates = gates + foz0
        f = jax.nn.sigmoid(gates[:, 0:H])
        o = jax.nn.sigmoid(gates[:, H:2 * H])
        z = jnp.tanh(gates[:, 2 * H:3 * H])
        c = gc * f + z * (1.0 - f)
        h = o * jnp.tanh(c)
        state_ref[pl.ds(i, 1), :, 0:H] = h[None]
        state_ref[pl.ds(i, 1), :, H:2 * H] = c[None]
        out_ref[pl.ds(i, 1), :, :] = h[None]
        return 0

    jax.lax.fori_loop(1, NODE_NUM, step, 0)


def kernel(tree_embedding, node_connection, node_mask, W_f, b_f, W_o, b_o,
           W_z, b_z, T_f, T_o, T_z, init_h, init_c):
    del node_mask
    H = HIDDEN_SZ
    emb0 = tree_embedding[:, 0, :]
    connT = jnp.swapaxes(node_connection, 0, 1)
    Wcat = jnp.concatenate([W_f.T, W_o.T, W_z.T], axis=1)
    bcat = jnp.concatenate([b_f, b_o, b_z]).reshape(1, 3 * H)
    init_hcT = jnp.swapaxes(jnp.concatenate([init_h, init_c], axis=2), 0, 1)

    nb = BATCH // BB
    grid = (nb,)
    out = pl.pallas_call(
        _kernel_body,
        grid=grid,
        in_specs=[
            pl.BlockSpec((BB, INPUT_SZ), lambda j: (j, 0)),
            pl.BlockSpec((NODE_NUM, BB), lambda j: (0, j),
                         memory_space=pltpu.SMEM),
            pl.BlockSpec((INPUT_SZ, 3 * H), lambda j: (0, 0)),
            pl.BlockSpec((1, 3 * H), lambda j: (0, 0)),
            pl.BlockSpec((H, H), lambda j: (0, 0)),
            pl.BlockSpec((H, H), lambda j: (0, 0)),
            pl.BlockSpec((H, H), lambda j: (0, 0)),
            pl.BlockSpec(memory_space=pltpu.MemorySpace.HBM),
        ],
        out_specs=pl.BlockSpec((NODE_NUM, BB, H), lambda j: (0, j, 0)),
        out_shape=jax.ShapeDtypeStruct((NODE_NUM, BATCH, H), jnp.float32),
        scratch_shapes=[
            pltpu.VMEM((NODE_NUM, BB, 2 * H), jnp.float32),
            pltpu.VMEM((1, BB, 2 * H), jnp.float32),
            pltpu.SemaphoreType.DMA,
        ],
    )(emb0, connT, Wcat, bcat, T_f, T_o, T_z, init_hcT)
    return jnp.swapaxes(out, 0, 1)


# speedup vs baseline: 11.6305x; 1.2306x over previous
import jax
import jax.numpy as jnp
from jax.experimental import pallas as pl
from jax.experimental.pallas import tpu as pltpu

BATCH = 1024
NODE_NUM = 128
INPUT_SZ = 128
HIDDEN_SZ = 128

BB = 256

_PREC_HI = jax.lax.Precision.HIGHEST
_PREC_STEP = jax.lax.Precision.HIGHEST


def _kernel_body(emb0_ref, connT_ref, Wcat_ref, bcat_ref, Tf_ref, To_ref,
                 Tz_ref, init_hT_ref, init_cT_ref, out_ref, c_ref, gh_ref,
                 gc_ref, sem_h, sem_c):
    H = HIDDEN_SZ
    j = pl.program_id(0)

    cp_h = pltpu.make_async_copy(init_hT_ref.at[:, pl.ds(j * BB, BB), :],
                                 out_ref, sem_h)
    cp_c = pltpu.make_async_copy(init_cT_ref.at[:, pl.ds(j * BB, BB), :],
                                 c_ref, sem_c)
    cp_h.start()
    cp_c.start()

    def tt(t_ref):
        t = t_ref[...]
        return jax.lax.dot_general(t, t, (((0,), (0,)), ((), ())),
                                   precision=_PREC_HI,
                                   preferred_element_type=jnp.float32)

    vcat = jnp.concatenate([tt(Tf_ref), tt(To_ref), tt(Tz_ref)], axis=1)

    foz0 = jax.lax.dot_general(emb0_ref[...], Wcat_ref[...],
                               (((1,), (0,)), ((), ())), precision=_PREC_HI,
                               preferred_element_type=jnp.float32)
    foz0 = foz0 + bcat_ref[...]
    f0 = foz0[:, 0:H]
    o0 = foz0[:, H:2 * H]
    z0 = foz0[:, 2 * H:3 * H]

    f = jax.nn.sigmoid(f0)
    o = jax.nn.sigmoid(o0)
    z = jnp.tanh(z0)
    c0 = z * (1.0 - f)
    h0 = o * jnp.tanh(c0)

    cp_h.wait()
    cp_c.wait()
    out_ref[pl.ds(0, 1), :, :] = h0[None]
    c_ref[pl.ds(0, 1), :, :] = c0[None]

    def step(i, _):
        for b in range(BB):
            p = connT_ref[i, b]
            gh_ref[:, pl.ds(b, 1), :] = out_ref[pl.ds(p, 1), pl.ds(b, 1), :]
            gc_ref[:, pl.ds(b, 1), :] = c_ref[pl.ds(p, 1), pl.ds(b, 1), :]
        gates = jax.lax.dot_general(gh_ref[0], vcat, (((1,), (0,)), ((), ())),
                                    precision=_PREC_STEP,
                                    preferred_element_type=jnp.float32)
        gates = gates + foz0
        f = jax.nn.sigmoid(gates[:, 0:H])
        o = jax.nn.sigmoid(gates[:, H:2 * H])
        z = jnp.tanh(gates[:, 2 * H:3 * H])
        c = gc_ref[0] * f + z * (1.0 - f)
        h = o * jnp.tanh(c)
        out_ref[pl.ds(i, 1), :, :] = h[None]
        c_ref[pl.ds(i, 1), :, :] = c[None]
        return 0

    jax.lax.fori_loop(1, NODE_NUM, step, 0)


def kernel(tree_embedding, node_connection, node_mask, W_f, b_f, W_o, b_o,
           W_z, b_z, T_f, T_o, T_z, init_h, init_c):
    del node_mask
    H = HIDDEN_SZ
    emb0 = tree_embedding[:, 0, :]
    connT = jnp.swapaxes(node_connection, 0, 1)
    Wcat = jnp.concatenate([W_f.T, W_o.T, W_z.T], axis=1)
    bcat = jnp.concatenate([b_f, b_o, b_z]).reshape(1, 3 * H)
    init_hT = jnp.swapaxes(init_h, 0, 1)
    init_cT = jnp.swapaxes(init_c, 0, 1)

    nb = BATCH // BB
    grid = (nb,)
    out = pl.pallas_call(
        _kernel_body,
        grid=grid,
        in_specs=[
            pl.BlockSpec((BB, INPUT_SZ), lambda j: (j, 0)),
            pl.BlockSpec((NODE_NUM, BB), lambda j: (0, j),
                         memory_space=pltpu.SMEM),
            pl.BlockSpec((INPUT_SZ, 3 * H), lambda j: (0, 0)),
            pl.BlockSpec((1, 3 * H), lambda j: (0, 0)),
            pl.BlockSpec((H, H), lambda j: (0, 0)),
            pl.BlockSpec((H, H), lambda j: (0, 0)),
            pl.BlockSpec((H, H), lambda j: (0, 0)),
            pl.BlockSpec(memory_space=pltpu.MemorySpace.HBM),
            pl.BlockSpec(memory_space=pltpu.MemorySpace.HBM),
        ],
        out_specs=pl.BlockSpec((NODE_NUM, BB, H), lambda j: (0, j, 0)),
        out_shape=jax.ShapeDtypeStruct((NODE_NUM, BATCH, H), jnp.float32),
        scratch_shapes=[
            pltpu.VMEM((NODE_NUM, BB, H), jnp.float32),
            pltpu.VMEM((1, BB, H), jnp.float32),
            pltpu.VMEM((1, BB, H), jnp.float32),
            pltpu.SemaphoreType.DMA,
            pltpu.SemaphoreType.DMA,
        ],
    )(emb0, connT, Wcat, bcat, T_f, T_o, T_z, init_hT, init_cT)
    return jnp.swapaxes(out, 0, 1)


# speedup vs baseline: 14.6514x; 1.2597x over previous
import jax
import jax.numpy as jnp
from jax.experimental import pallas as pl
from jax.experimental.pallas import tpu as pltpu

BATCH = 1024
NODE_NUM = 128
INPUT_SZ = 128
HIDDEN_SZ = 128

BB = 256

_PREC_HI = jax.lax.Precision.HIGHEST
_PREC_STEP = jax.lax.Precision.DEFAULT


def _kernel_body(emb0_ref, connT_ref, Wcat_ref, bcat_ref, Tf_ref, To_ref,
                 Tz_ref, init_hT_ref, init_cT_ref, out_ref, c_ref, gh_ref,
                 gc_ref, sem_h, sem_c):
    H = HIDDEN_SZ
    j = pl.program_id(0)

    cp_h = pltpu.make_async_copy(init_hT_ref.at[:, pl.ds(j * BB, BB), :],
                                 out_ref, sem_h)
    cp_c = pltpu.make_async_copy(init_cT_ref.at[:, pl.ds(j * BB, BB), :],
                                 c_ref, sem_c)
    cp_h.start()
    cp_c.start()

    def tt(t_ref):
        t = t_ref[...]
        return jax.lax.dot_general(t, t, (((0,), (0,)), ((), ())),
                                   precision=_PREC_HI,
                                   preferred_element_type=jnp.float32)

    vcat = jnp.concatenate([tt(Tf_ref), tt(To_ref), tt(Tz_ref)], axis=1)

    foz0 = jax.lax.dot_general(emb0_ref[...], Wcat_ref[...],
                               (((1,), (0,)), ((), ())), precision=_PREC_HI,
                               preferred_element_type=jnp.float32)
    foz0 = foz0 + bcat_ref[...]
    f0 = foz0[:, 0:H]
    o0 = foz0[:, H:2 * H]
    z0 = foz0[:, 2 * H:3 * H]

    f = jax.nn.sigmoid(f0)
    o = jax.nn.sigmoid(o0)
    z = jnp.tanh(z0)
    c0 = z * (1.0 - f)
    h0 = o * jnp.tanh(c0)

    cp_h.wait()
    cp_c.wait()
    out_ref[pl.ds(0, 1), :, :] = h0[None]
    c_ref[pl.ds(0, 1), :, :] = c0[None]

    def step(i, _):
        for b in range(BB):
            p = connT_ref[i, b]
            gh_ref[:, pl.ds(b, 1), :] = out_ref[pl.ds(p, 1), pl.ds(b, 1), :]
            gc_ref[:, pl.ds(b, 1), :] = c_ref[pl.ds(p, 1), pl.ds(b, 1), :]
        gates = jax.lax.dot_general(gh_ref[0], vcat, (((1,), (0,)), ((), ())),
                                    precision=_PREC_STEP,
                                    preferred_element_type=jnp.float32)
        gates = gates + foz0
        f = jax.nn.sigmoid(gates[:, 0:H])
        o = jax.nn.sigmoid(gates[:, H:2 * H])
        z = jnp.tanh(gates[:, 2 * H:3 * H])
        c = gc_ref[0] * f + z * (1.0 - f)
        h = o * jnp.tanh(c)
        out_ref[pl.ds(i, 1), :, :] = h[None]
        c_ref[pl.ds(i, 1), :, :] = c[None]
        return 0

    jax.lax.fori_loop(1, NODE_NUM, step, 0)


def kernel(tree_embedding, node_connection, node_mask, W_f, b_f, W_o, b_o,
           W_z, b_z, T_f, T_o, T_z, init_h, init_c):
    del node_mask
    H = HIDDEN_SZ
    emb0 = tree_embedding[:, 0, :]
    connT = jnp.swapaxes(node_connection, 0, 1)
    Wcat = jnp.concatenate([W_f.T, W_o.T, W_z.T], axis=1)
    bcat = jnp.concatenate([b_f, b_o, b_z]).reshape(1, 3 * H)
    init_hT = jnp.swapaxes(init_h, 0, 1)
    init_cT = jnp.swapaxes(init_c, 0, 1)

    nb = BATCH // BB
    grid = (nb,)
    out = pl.pallas_call(
        _kernel_body,
        grid=grid,
        in_specs=[
            pl.BlockSpec((BB, INPUT_SZ), lambda j: (j, 0)),
            pl.BlockSpec((NODE_NUM, BB), lambda j: (0, j),
                         memory_space=pltpu.SMEM),
            pl.BlockSpec((INPUT_SZ, 3 * H), lambda j: (0, 0)),
            pl.BlockSpec((1, 3 * H), lambda j: (0, 0)),
            pl.BlockSpec((H, H), lambda j: (0, 0)),
            pl.BlockSpec((H, H), lambda j: (0, 0)),
            pl.BlockSpec((H, H), lambda j: (0, 0)),
            pl.BlockSpec(memory_space=pltpu.MemorySpace.HBM),
            pl.BlockSpec(memory_space=pltpu.MemorySpace.HBM),
        ],
        out_specs=pl.BlockSpec((NODE_NUM, BB, H), lambda j: (0, j, 0)),
        out_shape=jax.ShapeDtypeStruct((NODE_NUM, BATCH, H), jnp.float32),
        scratch_shapes=[
            pltpu.VMEM((NODE_NUM, BB, H), jnp.float32),
            pltpu.VMEM((1, BB, H), jnp.float32),
            pltpu.VMEM((1, BB, H), jnp.float32),
            pltpu.SemaphoreType.DMA,
            pltpu.SemaphoreType.DMA,
        ],
    )(emb0, connT, Wcat, bcat, T_f, T_o, T_z, init_hT, init_cT)
    return jnp.swapaxes(out, 0, 1)
